# Initial kernel scaffold; baseline (speedup 1.0000x reference)
import jax
import jax.numpy as jnp
from jax.experimental import pallas as pl


def kernel(sparse_rep, en_token_ids_list):
    raise NotImplementedError("write your pallas kernel here")


# speedup vs baseline: 3.0873x; 3.0873x over previous
import functools

import jax
import jax.numpy as jnp
from jax import lax
from jax.experimental import pallas as pl
from jax.experimental.pallas import tpu as pltpu
from jax.experimental.pallas import tpu_sc as plsc

B = 128
V = 100000
T = 20
TP = 32
NB = 4096
TOPK = 100
NV = V // 16
ROWS_PER_W = 4
NBV = NB // 16


def _crossing_search(hist_ref, sum_ref, target):
    iota = lax.iota(jnp.int32, 16)

    def cond(carry):
        i, acc, found, b, c_ab, s_ab = carry
        return jnp.logical_and(jnp.logical_not(found), i < NBV)

    def body(carry):
        i, acc, found, b, c_ab, s_ab = carry
        vi = (NBV - 1) - i
        h = hist_ref[pl.ds(vi * 16, 16)]
        hr = lax.rev(h, (0,))
        c = plsc.cumsum(hr)
        tot = jnp.max(c)
        cross = acc + tot >= target
        m = (c + acc) >= target
        k = jnp.max(plsc.all_reduce_ffs(m))
        hk = jnp.sum(jnp.where(iota == k, hr, 0))
        ck = jnp.sum(jnp.where(iota == k, c, 0))
        b_new = jnp.where(cross, vi * 16 + (15 - k), b)
        c_new = jnp.where(cross, acc + ck - hk, c_ab)
        if sum_ref is not None:
            s = sum_ref[pl.ds(vi * 16, 16)]
            sr = lax.rev(s, (0,))
            part = jnp.sum(jnp.where(iota < k, sr, jnp.float32(0.0)))
            full = jnp.sum(sr)
            s_new = s_ab + jnp.where(cross, part, full)
        else:
            s_new = s_ab
        return (i + 1, acc + tot, cross, b_new, c_new, s_new)

    init = (jnp.int32(0), jnp.int32(0), jnp.bool_(False), jnp.int32(0),
            jnp.int32(0), jnp.float32(0.0))
    _, _, _, b, c_ab, s_ab = lax.while_loop(cond, body, init)
    return b, c_ab, s_ab


def _sc_body(rep_hbm, ids_hbm, tacts_hbm, neg_hbm,
             rowbuf, ids_v, tstage, negstage, hist, subcnt, subsum):
    wid = lax.axis_index("s") * 2 + lax.axis_index("c")
    iota = lax.iota(jnp.int32, 16)
    one16 = jnp.ones((16,), jnp.int32)
    zero16i = jnp.zeros((16,), jnp.int32)
    zero16f = jnp.zeros((16,), jnp.float32)

    for rr in range(ROWS_PER_W):
        r = wid * ROWS_PER_W + rr

        pltpu.sync_copy(rep_hbm.at[pl.ds(r * V, V)], rowbuf)
        pltpu.sync_copy(ids_hbm.at[pl.ds(r * TP, TP)], ids_v)

        i1 = ids_v[pl.ds(0, 16)]
        i2 = ids_v[pl.ds(16, 16)]
        m2 = iota < (T - 16)
        g1 = plsc.load_gather(rowbuf, [i1])
        g2 = plsc.load_gather(rowbuf, [i2])
        g2 = jnp.where(m2, g2, jnp.float32(1.0))
        tstage[pl.ds(0, 16)] = g1
        tstage[pl.ds(16, 16)] = g2
        pltpu.sync_copy(tstage, tacts_hbm.at[pl.ds(r * TP, TP)])
        neg1 = jnp.full((16,), -1.0, jnp.float32)
        plsc.store_scatter(rowbuf, [i1], neg1)
        plsc.store_scatter(rowbuf, [i2], neg1, mask=m2)

        def zero_body(i, _):
            hist[pl.ds(i * 16, 16)] = zero16i
            subcnt[pl.ds(i * 16, 16)] = zero16i
            subsum[pl.ds(i * 16, 16)] = zero16f
            return 0
        lax.fori_loop(0, NBV, zero_body, 0)

        def pass_a(j, _):
            x = rowbuf[pl.ds(j * 16, 16)]
            xf = x * jnp.float32(NB)
            bn = jnp.maximum(xf.astype(jnp.int32), 0)
            valid = x >= jnp.float32(0.0)
            plsc.addupdate_scatter(hist, [bn], one16, mask=valid)
            return 0
        lax.fori_loop(0, NV, pass_a, 0, unroll=8)

        b1, c1, _ = _crossing_search(hist, None, jnp.int32(TOPK))
        n1 = jnp.int32(TOPK) - c1

        def pass_b(j, accv):
            x = rowbuf[pl.ds(j * 16, 16)]
            xf = x * jnp.float32(NB)
            braw = xf.astype(jnp.int32)
            bn = jnp.maximum(braw, 0)
            valid = x >= jnp.float32(0.0)
            above = jnp.logical_and(valid, bn > b1)
            accv = accv + jnp.where(above, x, jnp.float32(0.0))
            eq = jnp.logical_and(valid, bn == b1)
            frac = xf - braw.astype(jnp.float32)
            sub = jnp.clip((frac * jnp.float32(NB)).astype(jnp.int32),
                           0, NB - 1)
            plsc.addupdate_scatter(subcnt, [sub], one16, mask=eq)
            plsc.addupdate_scatter(subsum, [sub], x, mask=eq)
            return accv
        accv = lax.fori_loop(0, NV, pass_b, zero16f, unroll=4)
        s_above = jnp.sum(accv)

        b2, c2, s2 = _crossing_search(subcnt, subsum, n1)
        n2 = (n1 - c2).astype(jnp.float32)
        inv_nb = jnp.float32(1.0 / NB)
        vhat = (b1.astype(jnp.float32)
                + (b2.astype(jnp.float32) + jnp.float32(0.5))
                * inv_nb) * inv_nb
        rowsum = s_above + s2 + n2 * vhat

        negstage[...] = jnp.full((16,), rowsum, jnp.float32)
        pltpu.sync_copy(negstage, neg_hbm.at[pl.ds(r * 16, 16)])


def _tc_body(tacts_ref, neg_ref, tl_ref, ml_ref, nl_ref):
    t = tacts_ref[...]
    lanes = lax.broadcasted_iota(jnp.int32, t.shape, 1)
    valid = lanes < T
    logt = -jnp.log(t + jnp.float32(1e-8))
    tl_ref[0, 0] = jnp.sum(jnp.where(valid, logt, 0.0)) / jnp.float32(B * T)
    marg = jnp.maximum(jnp.float32(1.0) - t, 0.0)
    ml_ref[0, 0] = jnp.sum(jnp.where(valid, marg, 0.0)) / jnp.float32(B * T)
    neg = neg_ref[...]
    lanes2 = lax.broadcasted_iota(jnp.int32, neg.shape, 1)
    nl_ref[0, 0] = (jnp.sum(jnp.where(lanes2 == 0, neg, 0.0))
                    / jnp.float32(B * TOPK))


@jax.jit
def kernel(sparse_rep, en_token_ids_list):
    rep1d = sparse_rep.reshape(-1)
    ids = en_token_ids_list.astype(jnp.int32)
    ids_p = jnp.pad(ids, ((0, 0), (0, TP - T))).reshape(-1)

    mesh = plsc.VectorSubcoreMesh(core_axis_name="c", subcore_axis_name="s")
    sc = pl.kernel(
        _sc_body,
        out_type=(
            jax.ShapeDtypeStruct((B * TP,), jnp.float32),
            jax.ShapeDtypeStruct((B * 16,), jnp.float32),
        ),
        mesh=mesh,
        compiler_params=pltpu.CompilerParams(needs_layout_passes=False),
        scratch_types=[
            pltpu.VMEM((V,), jnp.float32),
            pltpu.VMEM((TP,), jnp.int32),
            pltpu.VMEM((TP,), jnp.float32),
            pltpu.VMEM((16,), jnp.float32),
            pltpu.VMEM((NB,), jnp.int32),
            pltpu.VMEM((NB,), jnp.int32),
            pltpu.VMEM((NB,), jnp.float32),
        ],
    )
    tacts_f, neg_f = sc(rep1d, ids_p)
    tacts = tacts_f.reshape(B, TP)
    negs = neg_f.reshape(B, 16)

    tl, ml, nl = pl.pallas_call(
        _tc_body,
        out_shape=(jax.ShapeDtypeStruct((1, 1), jnp.float32),) * 3,
        out_specs=(pl.BlockSpec(memory_space=pltpu.SMEM),) * 3,
    )(tacts, negs)
    return (tl[0, 0], ml[0, 0], nl[0, 0])


# speedup vs baseline: 8.7129x; 2.8221x over previous
import functools

import jax
import jax.numpy as jnp
from jax import lax
from jax.experimental import pallas as pl
from jax.experimental.pallas import tpu as pltpu
from jax.experimental.pallas import tpu_sc as plsc

B = 128
V = 100000
T = 20
TP = 32
NB = 4096
TOPK = 100
NV = V // 16
ROWS_PER_W = 4
NBV = NB // 16


def _crossing_search(hist_ref, sum_ref, target):
    iota = lax.iota(jnp.int32, 16)

    def cond(carry):
        i, acc, found, b, c_ab, s_ab = carry
        return jnp.logical_and(jnp.logical_not(found), i < NBV)

    def body(carry):
        i, acc, found, b, c_ab, s_ab = carry
        vi = (NBV - 1) - i
        h = hist_ref[pl.ds(vi * 16, 16)]
        hr = lax.rev(h, (0,))
        c = plsc.cumsum(hr)
        tot = jnp.max(c)
        cross = acc + tot >= target
        m = (c + acc) >= target
        k = jnp.max(plsc.all_reduce_ffs(m))
        hk = jnp.sum(jnp.where(iota == k, hr, 0))
        ck = jnp.sum(jnp.where(iota == k, c, 0))
        b_new = jnp.where(cross, vi * 16 + (15 - k), b)
        c_new = jnp.where(cross, acc + ck - hk, c_ab)
        if sum_ref is not None:
            s = sum_ref[pl.ds(vi * 16, 16)]
            sr = lax.rev(s, (0,))
            part = jnp.sum(jnp.where(iota < k, sr, jnp.float32(0.0)))
            full = jnp.sum(sr)
            s_new = s_ab + jnp.where(cross, part, full)
        else:
            s_new = s_ab
        return (i + 1, acc + tot, cross, b_new, c_new, s_new)

    init = (jnp.int32(0), jnp.int32(0), jnp.bool_(False), jnp.int32(0),
            jnp.int32(0), jnp.float32(0.0))
    _, _, _, b, c_ab, s_ab = lax.while_loop(cond, body, init)
    return b, c_ab, s_ab


def _sc_body(rep_hbm, ids_hbm, tacts_hbm, neg_hbm,
             rowbuf, ids_v, tstage, negstage, hist, subcnt, subsum):
    wid = lax.axis_index("s") * 2 + lax.axis_index("c")
    iota = lax.iota(jnp.int32, 16)
    one16 = jnp.ones((16,), jnp.int32)
    zero16i = jnp.zeros((16,), jnp.int32)
    zero16f = jnp.zeros((16,), jnp.float32)

    for rr in range(ROWS_PER_W):
        r = wid * ROWS_PER_W + rr

        pltpu.sync_copy(rep_hbm.at[pl.ds(r * V, V)], rowbuf)
        pltpu.sync_copy(ids_hbm.at[pl.ds(r * TP, TP)], ids_v)

        i1 = ids_v[pl.ds(0, 16)]
        i2 = ids_v[pl.ds(16, 16)]
        m2 = iota < (T - 16)
        g1 = plsc.load_gather(rowbuf, [i1])
        g2 = plsc.load_gather(rowbuf, [i2])
        g2 = jnp.where(m2, g2, jnp.float32(1.0))
        tstage[pl.ds(0, 16)] = g1
        tstage[pl.ds(16, 16)] = g2
        pltpu.sync_copy(tstage, tacts_hbm.at[pl.ds(r * TP, TP)])
        neg1 = jnp.full((16,), -1.0, jnp.float32)
        plsc.store_scatter(rowbuf, [i1], neg1)
        plsc.store_scatter(rowbuf, [i2], neg1, mask=m2)

        @plsc.parallel_loop(0, NBV, unroll=8)
        def zero_body(i):
            hist[pl.ds(i * 16, 16)] = zero16i
            subcnt[pl.ds(i * 16, 16)] = zero16i
            subsum[pl.ds(i * 16, 16)] = zero16f
        subsum[pl.ds(NB, 16)] = zero16f

        @plsc.parallel_loop(0, NV, unroll=10)
        def pass_a(j):
            x = rowbuf[pl.ds(j * 16, 16)]
            xf = x * jnp.float32(NB)
            bn = jnp.maximum(xf.astype(jnp.int32), 0)
            valid = x >= jnp.float32(0.0)
            plsc.addupdate_scatter(hist, [bn], one16, mask=valid)

        b1, c1, _ = _crossing_search(hist, None, jnp.int32(TOPK))
        n1 = jnp.int32(TOPK) - c1

        lane_slot = iota + jnp.int32(NB)

        @plsc.parallel_loop(0, NV, unroll=6)
        def pass_b(j):
            x = rowbuf[pl.ds(j * 16, 16)]
            xf = x * jnp.float32(NB)
            braw = xf.astype(jnp.int32)
            bn = jnp.maximum(braw, 0)
            valid = x >= jnp.float32(0.0)
            above = jnp.logical_and(valid, bn > b1)
            eq = jnp.logical_and(valid, bn == b1)
            frac = xf - braw.astype(jnp.float32)
            sub = jnp.clip((frac * jnp.float32(NB)).astype(jnp.int32),
                           0, NB - 1)
            idx = jnp.where(above, lane_slot, sub)
            plsc.addupdate_scatter(subcnt, [sub], one16, mask=eq)
            plsc.addupdate_scatter(subsum, [idx], x,
                                   mask=jnp.logical_or(above, eq))

        s_above = jnp.sum(subsum[pl.ds(NB, 16)])

        b2, c2, s2 = _crossing_search(subcnt, subsum, n1)
        n2 = (n1 - c2).astype(jnp.float32)
        inv_nb = jnp.float32(1.0 / NB)
        vhat = (b1.astype(jnp.float32)
                + (b2.astype(jnp.float32) + jnp.float32(0.5))
                * inv_nb) * inv_nb
        rowsum = s_above + s2 + n2 * vhat

        negstage[...] = jnp.full((16,), rowsum, jnp.float32)
        pltpu.sync_copy(negstage, neg_hbm.at[pl.ds(r * 16, 16)])


def _tc_body(tacts_ref, neg_ref, tl_ref, ml_ref, nl_ref):
    t = tacts_ref[...]
    lanes = lax.broadcasted_iota(jnp.int32, t.shape, 1)
    valid = lanes < T
    logt = -jnp.log(t + jnp.float32(1e-8))
    tl_ref[0, 0] = jnp.sum(jnp.where(valid, logt, 0.0)) / jnp.float32(B * T)
    marg = jnp.maximum(jnp.float32(1.0) - t, 0.0)
    ml_ref[0, 0] = jnp.sum(jnp.where(valid, marg, 0.0)) / jnp.float32(B * T)
    neg = neg_ref[...]
    lanes2 = lax.broadcasted_iota(jnp.int32, neg.shape, 1)
    nl_ref[0, 0] = (jnp.sum(jnp.where(lanes2 == 0, neg, 0.0))
                    / jnp.float32(B * TOPK))


@jax.jit
def kernel(sparse_rep, en_token_ids_list):
    rep1d = sparse_rep.reshape(-1)
    ids = en_token_ids_list.astype(jnp.int32)
    ids_p = jnp.pad(ids, ((0, 0), (0, TP - T))).reshape(-1)

    mesh = plsc.VectorSubcoreMesh(core_axis_name="c", subcore_axis_name="s")
    sc = pl.kernel(
        _sc_body,
        out_type=(
            jax.ShapeDtypeStruct((B * TP,), jnp.float32),
            jax.ShapeDtypeStruct((B * 16,), jnp.float32),
        ),
        mesh=mesh,
        compiler_params=pltpu.CompilerParams(needs_layout_passes=False),
        scratch_types=[
            pltpu.VMEM((V,), jnp.float32),
            pltpu.VMEM((TP,), jnp.int32),
            pltpu.VMEM((TP,), jnp.float32),
            pltpu.VMEM((16,), jnp.float32),
            pltpu.VMEM((NB,), jnp.int32),
            pltpu.VMEM((NB,), jnp.int32),
            pltpu.VMEM((NB + 16,), jnp.float32),
        ],
    )
    tacts_f, neg_f = sc(rep1d, ids_p)
    tacts = tacts_f.reshape(B, TP)
    negs = neg_f.reshape(B, 16)

    tl, ml, nl = pl.pallas_call(
        _tc_body,
        out_shape=(jax.ShapeDtypeStruct((1, 1), jnp.float32),) * 3,
        out_specs=(pl.BlockSpec(memory_space=pltpu.SMEM),) * 3,
    )(tacts, negs)
    return (tl[0, 0], ml[0, 0], nl[0, 0])


# speedup vs baseline: 13.4169x; 1.5399x over previous
import functools

import jax
import jax.numpy as jnp
from jax import lax
from jax.experimental import pallas as pl
from jax.experimental.pallas import tpu as pltpu
from jax.experimental.pallas import tpu_sc as plsc

B = 128
V = 100000
T = 20
TP = 32
NB = 4096
TOPK = 100
NV = V // 16
ROWS_PER_W = 4
NBV = NB // 16
TAU = 255.0 / 256.0
KTAIL = float(NB * 256)


def _crossing_search(hist_ref, sum_ref, target):
    iota = lax.iota(jnp.int32, 16)

    def cond(carry):
        i, acc, found, b, c_ab, s_ab = carry
        return jnp.logical_and(jnp.logical_not(found), i < NBV)

    def body(carry):
        i, acc, found, b, c_ab, s_ab = carry
        vi = (NBV - 1) - i
        h = hist_ref[pl.ds(vi * 16, 16)]
        hr = lax.rev(h, (0,))
        c = plsc.cumsum(hr)
        tot = jnp.max(c)
        cross = acc + tot >= target
        m = (c + acc) >= target
        k = jnp.max(plsc.all_reduce_ffs(m))
        hk = jnp.sum(jnp.where(iota == k, hr, 0))
        ck = jnp.sum(jnp.where(iota == k, c, 0))
        b_new = jnp.where(cross, vi * 16 + (15 - k), b)
        c_new = jnp.where(cross, acc + ck - hk, c_ab)
        if sum_ref is not None:
            s = sum_ref[pl.ds(vi * 16, 16)]
            sr = lax.rev(s, (0,))
            part = jnp.sum(jnp.where(iota < k, sr, jnp.float32(0.0)))
            full = jnp.sum(sr)
            s_new = s_ab + jnp.where(cross, part, full)
        else:
            s_new = s_ab
        return (i + 1, acc + tot, cross, b_new, c_new, s_new)

    init = (jnp.int32(0), jnp.int32(0), jnp.bool_(False), jnp.int32(0),
            jnp.int32(0), jnp.float32(0.0))
    _, _, found, b, c_ab, s_ab = lax.while_loop(cond, body, init)
    return found, b, c_ab, s_ab


def _sc_body(rep_hbm, ids_hbm, tacts_hbm, neg_hbm,
             rowbuf, ids_v, tstage, negstage, slowres, hist, subcnt, subsum):
    wid = lax.axis_index("s") * 2 + lax.axis_index("c")
    iota = lax.iota(jnp.int32, 16)
    one16 = jnp.ones((16,), jnp.int32)
    zero16i = jnp.zeros((16,), jnp.int32)
    zero16f = jnp.zeros((16,), jnp.float32)

    for rr in range(ROWS_PER_W):
        r = wid * ROWS_PER_W + rr

        pltpu.sync_copy(rep_hbm.at[pl.ds(r * V, V)], rowbuf)
        pltpu.sync_copy(ids_hbm.at[pl.ds(r * TP, TP)], ids_v)

        i1 = ids_v[pl.ds(0, 16)]
        i2 = ids_v[pl.ds(16, 16)]
        m2 = iota < (T - 16)
        g1 = plsc.load_gather(rowbuf, [i1])
        g2 = plsc.load_gather(rowbuf, [i2])
        g2 = jnp.where(m2, g2, jnp.float32(1.0))
        tstage[pl.ds(0, 16)] = g1
        tstage[pl.ds(16, 16)] = g2
        pltpu.sync_copy(tstage, tacts_hbm.at[pl.ds(r * TP, TP)])
        neg1 = jnp.full((16,), -1.0, jnp.float32)
        plsc.store_scatter(rowbuf, [i1], neg1)
        plsc.store_scatter(rowbuf, [i2], neg1, mask=m2)

        @plsc.parallel_loop(0, NBV, unroll=8)
        def zero_fast(i):
            hist[pl.ds(i * 16, 16)] = zero16i
            subsum[pl.ds(i * 16, 16)] = zero16f

        @plsc.parallel_loop(0, NV, unroll=8)
        def fast_pass(j):
            x = rowbuf[pl.ds(j * 16, 16)]
            d = x - jnp.float32(TAU)
            idx = jnp.clip((d * jnp.float32(KTAIL)).astype(jnp.int32),
                           0, NB - 1)
            m = x >= jnp.float32(TAU)
            plsc.addupdate_scatter(hist, [idx], one16, mask=m)
            plsc.addupdate_scatter(subsum, [idx], x, mask=m)

        fast_ok, bf, cf, sf = _crossing_search(hist, subsum, jnp.int32(TOPK))
        vhat_f = (jnp.float32(TAU)
                  + (bf.astype(jnp.float32) + jnp.float32(0.5))
                  * jnp.float32(1.0 / KTAIL))
        rowsum_fast = sf + (jnp.int32(TOPK) - cf).astype(jnp.float32) * vhat_f

        slowres[...] = zero16f

        @pl.when(jnp.logical_not(fast_ok))
        def slow_path():
            @plsc.parallel_loop(0, NBV, unroll=8)
            def zero_body(i):
                hist[pl.ds(i * 16, 16)] = zero16i
                subcnt[pl.ds(i * 16, 16)] = zero16i
                subsum[pl.ds(i * 16, 16)] = zero16f
            subsum[pl.ds(NB, 16)] = zero16f

            @plsc.parallel_loop(0, NV, unroll=10)
            def pass_a(j):
                x = rowbuf[pl.ds(j * 16, 16)]
                xf = x * jnp.float32(NB)
                bn = jnp.maximum(xf.astype(jnp.int32), 0)
                valid = x >= jnp.float32(0.0)
                plsc.addupdate_scatter(hist, [bn], one16, mask=valid)

            _, b1, c1, _ = _crossing_search(hist, None, jnp.int32(TOPK))
            n1 = jnp.int32(TOPK) - c1

            lane_slot = iota + jnp.int32(NB)

            @plsc.parallel_loop(0, NV, unroll=6)
            def pass_b(j):
                x = rowbuf[pl.ds(j * 16, 16)]
                xf = x * jnp.float32(NB)
                braw = xf.astype(jnp.int32)
                bn = jnp.maximum(braw, 0)
                valid = x >= jnp.float32(0.0)
                above = jnp.logical_and(valid, bn > b1)
                eq = jnp.logical_and(valid, bn == b1)
                frac = xf - braw.astype(jnp.float32)
                sub = jnp.clip((frac * jnp.float32(NB)).astype(jnp.int32),
                               0, NB - 1)
                idx = jnp.where(above, lane_slot, sub)
                plsc.addupdate_scatter(subcnt, [sub], one16, mask=eq)
                plsc.addupdate_scatter(subsum, [idx], x,
                                       mask=jnp.logical_or(above, eq))

            s_above = jnp.sum(subsum[pl.ds(NB, 16)])

            _, b2, c2, s2 = _crossing_search(subcnt, subsum, n1)
            n2 = (n1 - c2).astype(jnp.float32)
            inv_nb = jnp.float32(1.0 / NB)
            vhat = (b1.astype(jnp.float32)
                    + (b2.astype(jnp.float32) + jnp.float32(0.5))
                    * inv_nb) * inv_nb
            slowres[...] = jnp.full((16,), s_above + s2 + n2 * vhat,
                                    jnp.float32)

        rowsum = jnp.where(fast_ok, rowsum_fast, jnp.max(slowres[...]))

        negstage[...] = jnp.full((16,), rowsum, jnp.float32)
        pltpu.sync_copy(negstage, neg_hbm.at[pl.ds(r * 16, 16)])


def _tc_body(tacts_ref, neg_ref, tl_ref, ml_ref, nl_ref):
    t = tacts_ref[...]
    lanes = lax.broadcasted_iota(jnp.int32, t.shape, 1)
    valid = lanes < T
    logt = -jnp.log(t + jnp.float32(1e-8))
    tl_ref[0, 0] = jnp.sum(jnp.where(valid, logt, 0.0)) / jnp.float32(B * T)
    marg = jnp.maximum(jnp.float32(1.0) - t, 0.0)
    ml_ref[0, 0] = jnp.sum(jnp.where(valid, marg, 0.0)) / jnp.float32(B * T)
    neg = neg_ref[...]
    lanes2 = lax.broadcasted_iota(jnp.int32, neg.shape, 1)
    nl_ref[0, 0] = (jnp.sum(jnp.where(lanes2 == 0, neg, 0.0))
                    / jnp.float32(B * TOPK))


@jax.jit
def kernel(sparse_rep, en_token_ids_list):
    rep1d = sparse_rep.reshape(-1)
    ids = en_token_ids_list.astype(jnp.int32)
    ids_p = jnp.pad(ids, ((0, 0), (0, TP - T))).reshape(-1)

    mesh = plsc.VectorSubcoreMesh(core_axis_name="c", subcore_axis_name="s")
    sc = pl.kernel(
        _sc_body,
        out_type=(
            jax.ShapeDtypeStruct((B * TP,), jnp.float32),
            jax.ShapeDtypeStruct((B * 16,), jnp.float32),
        ),
        mesh=mesh,
        compiler_params=pltpu.CompilerParams(needs_layout_passes=False),
        scratch_types=[
            pltpu.VMEM((V,), jnp.float32),
            pltpu.VMEM((TP,), jnp.int32),
            pltpu.VMEM((TP,), jnp.float32),
            pltpu.VMEM((16,), jnp.float32),
            pltpu.VMEM((16,), jnp.float32),
            pltpu.VMEM((NB,), jnp.int32),
            pltpu.VMEM((NB,), jnp.int32),
            pltpu.VMEM((NB + 16,), jnp.float32),
        ],
    )
    tacts_f, neg_f = sc(rep1d, ids_p)
    tacts = tacts_f.reshape(B, TP)
    negs = neg_f.reshape(B, 16)

    tl, ml, nl = pl.pallas_call(
        _tc_body,
        out_shape=(jax.ShapeDtypeStruct((1, 1), jnp.float32),) * 3,
        out_specs=(pl.BlockSpec(memory_space=pltpu.SMEM),) * 3,
    )(tacts, negs)
    return (tl[0, 0], ml[0, 0], nl[0, 0])
